# baseline (speedup 1.0000x reference)
import functools
import math

import jax
import jax.numpy as jnp
from jax import lax
from jax.experimental import pallas as pl
from jax.experimental.pallas import tpu as pltpu
from jax.experimental.pallas import tpu_sc as plsc

_B = 4096
_F = 26
_D = 128
_V = 26000
_ROWS = _B * _F

_NC = 2
_NS = 16
_NW = _NC * _NS
_RPW = _ROWS // _NW
_CHUNK = 128
_NCHUNKS = _RPW // _CHUNK
_NBUF = 7

_MLP_ROWS = 5200
_MLP_GRID = _V // _MLP_ROWS

_INV_SQRT2 = 1.0 / math.sqrt(2.0)


def _mlp_body(x_ref, w1_ref, b1_ref, w2_ref, b2_ref, o_ref):
    x = x_ref[...]
    h = jnp.dot(x, w1_ref[...], preferred_element_type=jnp.float32) + b1_ref[...]
    h = 0.5 * h * (1.0 + lax.erf(h * _INV_SQRT2))
    o_ref[...] = jnp.dot(h, w2_ref[...], preferred_element_type=jnp.float32) + b2_ref[...]


def _mlp_table(attr_emb, W1, b1, W2, b2):
    return pl.pallas_call(
        _mlp_body,
        grid=(_MLP_GRID,),
        in_specs=[
            pl.BlockSpec((_MLP_ROWS, _D), lambda i: (i, 0)),
            pl.BlockSpec((_D, _D), lambda i: (0, 0)),
            pl.BlockSpec((1, _D), lambda i: (0, 0)),
            pl.BlockSpec((_D, _D), lambda i: (0, 0)),
            pl.BlockSpec((1, _D), lambda i: (0, 0)),
        ],
        out_specs=pl.BlockSpec((_MLP_ROWS, _D), lambda i: (i, 0)),
        out_shape=jax.ShapeDtypeStruct((_V, _D), jnp.float32),
    )(attr_emb, W1, b1[None, :], W2, b2[None, :])


@functools.lru_cache(maxsize=1)
def _sc_gather_kernel():
    @functools.partial(
        pl.kernel,
        out_type=jax.ShapeDtypeStruct((_ROWS, _D), jnp.float32),
        mesh=plsc.VectorSubcoreMesh(core_axis_name="c", subcore_axis_name="s"),
        scratch_types=[
            pltpu.VMEM((_NCHUNKS, _CHUNK), jnp.int32),
            pltpu.VMEM((_NBUF, _CHUNK, _D), jnp.float32),
            pltpu.SemaphoreType.DMA((_NBUF,)),
            pltpu.SemaphoreType.DMA((_NBUF,)),
        ],
    )
    def _sc_gather(table_hbm, idx_hbm, out_hbm, idx_v, bufs, gsems, ssems):
        wid = lax.axis_index("s") * _NC + lax.axis_index("c")
        base = wid * _RPW
        pltpu.sync_copy(idx_hbm.at[wid], idx_v)

        for b in range(_NBUF):
            pltpu.async_copy(table_hbm.at[idx_v.at[b]], bufs.at[b], gsems.at[b])

        def body(j, carry):
            b = lax.rem(j, _NBUF)
            pltpu.make_async_copy(
                table_hbm.at[pl.ds(0, _CHUNK)], bufs.at[b], gsems.at[b]
            ).wait()
            pltpu.async_copy(
                bufs.at[b], out_hbm.at[pl.ds(base + j * _CHUNK, _CHUNK)], ssems.at[b]
            )

            @pl.when(j + _NBUF < _NCHUNKS)
            def _():
                pltpu.make_async_copy(
                    table_hbm.at[pl.ds(0, _CHUNK)], bufs.at[b], ssems.at[b]
                ).wait()
                pltpu.async_copy(
                    table_hbm.at[idx_v.at[j + _NBUF]], bufs.at[b], gsems.at[b]
                )

            return carry

        lax.fori_loop(0, _NCHUNKS, body, 0)

        for b in range(_NBUF):
            pltpu.make_async_copy(
                table_hbm.at[pl.ds(0, _CHUNK)], bufs.at[b], ssems.at[b]
            ).wait()

    return _sc_gather


def kernel(attrs, attr_emb, W1, b1, W2, b2):
    shift = (jnp.arange(_F, dtype=attrs.dtype) * 1000)[:, None]
    idx = (attrs.T + shift).reshape(_NW, _NCHUNKS, _CHUNK)
    out_table = _mlp_table(attr_emb, W1, b1, W2, b2)
    out_flat = _sc_gather_kernel()(out_table, idx)
    return out_flat.reshape(_F, _B, _D).transpose(1, 0, 2)


# speedup vs baseline: 1.0103x; 1.0103x over previous
import functools
import math

import jax
import jax.numpy as jnp
from jax import lax
from jax.experimental import pallas as pl
from jax.experimental.pallas import tpu as pltpu
from jax.experimental.pallas import tpu_sc as plsc

_B = 4096
_F = 26
_D = 128
_V = 26000
_ROWS = _B * _F

_NC = 2
_NS = 16
_NW = _NC * _NS
_RPW = _ROWS // _NW
_CHUNK = 128
_NCHUNKS = _RPW // _CHUNK
_NBUF = 6

_MLP_ROWS = 5200
_MLP_GRID = _V // _MLP_ROWS

_INV_SQRT2 = 1.0 / math.sqrt(2.0)


def _mlp_body(x_ref, w1_ref, b1_ref, w2_ref, b2_ref, o_ref):
    x = x_ref[...]
    h = jnp.dot(x, w1_ref[...], preferred_element_type=jnp.float32) + b1_ref[...]
    h = 0.5 * h * (1.0 + lax.erf(h * _INV_SQRT2))
    o_ref[...] = jnp.dot(h, w2_ref[...], preferred_element_type=jnp.float32) + b2_ref[...]


def _mlp_table(attr_emb, W1, b1, W2, b2):
    return pl.pallas_call(
        _mlp_body,
        grid=(_MLP_GRID,),
        in_specs=[
            pl.BlockSpec((_MLP_ROWS, _D), lambda i: (i, 0)),
            pl.BlockSpec((_D, _D), lambda i: (0, 0)),
            pl.BlockSpec((1, _D), lambda i: (0, 0)),
            pl.BlockSpec((_D, _D), lambda i: (0, 0)),
            pl.BlockSpec((1, _D), lambda i: (0, 0)),
        ],
        out_specs=pl.BlockSpec((_MLP_ROWS, _D), lambda i: (i, 0)),
        out_shape=jax.ShapeDtypeStruct((_V, _D), jnp.float32),
    )(attr_emb, W1, b1[None, :], W2, b2[None, :])


@functools.lru_cache(maxsize=1)
def _sc_gather_kernel():
    @functools.partial(
        pl.kernel,
        out_type=jax.ShapeDtypeStruct((_ROWS, _D), jnp.float32),
        mesh=plsc.VectorSubcoreMesh(core_axis_name="c", subcore_axis_name="s"),
        scratch_types=[
            pltpu.VMEM((_NCHUNKS, _CHUNK), jnp.int32),
            pltpu.VMEM((_NBUF, _CHUNK, _D), jnp.float32),
            pltpu.SemaphoreType.DMA((_NBUF,)),
            pltpu.SemaphoreType.DMA((_NBUF,)),
        ],
    )
    def _sc_gather(table_hbm, idx_hbm, out_hbm, idx_v, bufs, gsems, ssems):
        wid = lax.axis_index("s") * _NC + lax.axis_index("c")
        base = wid * _RPW
        pltpu.sync_copy(idx_hbm.at[wid], idx_v)

        for b in range(_NBUF):
            pltpu.async_copy(table_hbm.at[idx_v.at[b]], bufs.at[b], gsems.at[b])

        def body(j, carry):
            b = lax.rem(j, _NBUF)
            pltpu.make_async_copy(
                table_hbm.at[pl.ds(0, _CHUNK)], bufs.at[b], gsems.at[b]
            ).wait()
            pltpu.async_copy(
                bufs.at[b], out_hbm.at[pl.ds(base + j * _CHUNK, _CHUNK)], ssems.at[b]
            )

            @pl.when(j + _NBUF < _NCHUNKS)
            def _():
                pltpu.make_async_copy(
                    table_hbm.at[pl.ds(0, _CHUNK)], bufs.at[b], ssems.at[b]
                ).wait()
                pltpu.async_copy(
                    table_hbm.at[idx_v.at[j + _NBUF]], bufs.at[b], gsems.at[b]
                )

            return carry

        lax.fori_loop(0, _NCHUNKS, body, 0)

        for b in range(_NBUF):
            pltpu.make_async_copy(
                table_hbm.at[pl.ds(0, _CHUNK)], bufs.at[b], ssems.at[b]
            ).wait()

    return _sc_gather


def kernel(attrs, attr_emb, W1, b1, W2, b2):
    shift = (jnp.arange(_F, dtype=attrs.dtype) * 1000)[:, None]
    idx = (attrs.T + shift).reshape(_NW, _NCHUNKS, _CHUNK)
    out_table = _mlp_table(attr_emb, W1, b1, W2, b2)
    out_flat = _sc_gather_kernel()(out_table, idx)
    return out_flat.reshape(_F, _B, _D).transpose(1, 0, 2)
